# baseline (speedup 1.0000x reference)
import functools

import jax
import jax.numpy as jnp
from jax.experimental import pallas as pl
from jax.experimental.pallas import tpu as pltpu
from jax.experimental.pallas import tpu_sc as plsc

B = 400
E = 256
TOTAL = B * (B - 1) // 2
H = 512
R = 2280
GRID = TOTAL // R

_bf = jnp.bfloat16


_SC_NW = 32
_SC_Q = 2496
_SC_C = 96


def _sc_copy_body(g_hbm, out_hbm, bufs, rsem, wsem):
    wid = jax.lax.axis_index("s") * 2 + jax.lax.axis_index("c")
    base = wid * _SC_Q
    nch = _SC_Q // _SC_C

    def off_of(k):
        return jnp.minimum(base + k * _SC_C, TOTAL - _SC_C)

    def rd(k, b):
        return pltpu.make_async_copy(
            g_hbm.at[pl.ds(off_of(k), _SC_C), :, :], bufs.at[b], rsem.at[b])

    def wr(k, b):
        return pltpu.make_async_copy(
            bufs.at[b], out_hbm.at[pl.ds(off_of(k), _SC_C), :, :], wsem.at[b])

    rd(0, 0).start()

    @pl.loop(0, nch)
    def _chunk(k):
        b = jax.lax.rem(k, 2)
        rd(k, b).wait()
        wr(k, b).start()

        @pl.when(k + 1 < nch)
        def _next():
            @pl.when(k >= 1)
            def _drain():
                wr(k - 1, 1 - b).wait()
            rd(k + 1, 1 - b).start()

    wr(nch - 2, jnp.int32(nch % 2)).wait()
    wr(nch - 1, jnp.int32((nch - 1) % 2)).wait()


def _sc_copy(graph_rep):
    return pl.kernel(
        _sc_copy_body,
        out_type=jax.ShapeDtypeStruct((TOTAL, 2, E), jnp.float32),
        mesh=plsc.VectorSubcoreMesh(core_axis_name="c", subcore_axis_name="s"),
        scratch_types=[
            pltpu.VMEM((2, _SC_C, 2, E), jnp.float32),
            pltpu.SemaphoreType.DMA((2,)),
            pltpu.SemaphoreType.DMA((2,)),
        ],
    )(graph_rep)


def _ln(x, g, b, eps=1e-5):
    mu = jnp.mean(x, axis=-1, keepdims=True)
    d = x - mu
    var = jnp.mean(d * d, axis=-1, keepdims=True)
    return d * jax.lax.rsqrt(var + eps) * g + b


def _mlp_kernel(cand_ref, g_ref, starts_ref, ends_ref, put_ref,
                w1_ref, b1_ref, g1_ref, bb1_ref,
                w2_ref, b2_ref, g2_ref, bb2_ref,
                w3_ref, b3_ref,
                out_ref,
                a_scr, acc_scr, cnt_scr):
    i = pl.program_id(0)

    @pl.when(i == 0)
    def _init():
        a_scr[...] = (jnp.dot(cand_ref[...].astype(_bf), w1_ref[0:E, :],
                              preferred_element_type=jnp.float32)
                      + b1_ref[...]).astype(_bf)
        acc_scr[...] = jnp.zeros_like(acc_scr)
        cnt_scr[...] = jnp.zeros_like(cnt_scr)

    iota_b = jax.lax.broadcasted_iota(jnp.int32, (1, B), 1)
    t = i * R + jax.lax.broadcasted_iota(jnp.int32, (R, 1), 0)
    ohb = jnp.logical_and(t >= starts_ref[...],
                          t < ends_ref[...]).astype(_bf)
    g2b = g_ref[...].astype(_bf)
    h = jnp.dot(ohb, a_scr[...], preferred_element_type=jnp.float32)
    h += jnp.dot(g2b, w1_ref[E:, :], preferred_element_type=jnp.float32)
    h = jax.nn.relu(_ln(h, g1_ref[...], bb1_ref[...]))

    h = jnp.dot(h.astype(_bf), w2_ref[...],
                preferred_element_type=jnp.float32) + b2_ref[...]
    h = jax.nn.relu(_ln(h, g2_ref[...], bb2_ref[...]))

    oh_put = (put_ref[...] == iota_b).astype(_bf)
    acc_scr[...] += jax.lax.dot_general(
        oh_put, h.astype(_bf), (((0,), (0,)), ((), ())),
        preferred_element_type=jnp.float32)
    cnt_scr[...] += jax.lax.dot_general(
        oh_put, jnp.ones((R, 1), _bf), (((0,), (0,)), ((), ())),
        preferred_element_type=jnp.float32)

    @pl.when(i == GRID - 1)
    def _fin():
        upd = jnp.dot(acc_scr[...].astype(_bf), w3_ref[...],
                      preferred_element_type=jnp.float32)
        out_ref[...] = cand_ref[...] + upd + cnt_scr[...] * b3_ref[...]


@jax.jit
def _run(candidate_rep, graph_rep, starts, ends, put2d,
         W1, b1, g1, bb1, W2, b2, g2, bb2, W3, b3):
    def full(*s):
        return pl.BlockSpec(s, lambda *_: tuple(0 for _ in s))
    return pl.pallas_call(
        _mlp_kernel,
        grid=(GRID,),
        in_specs=[
            full(B, E),
            pl.BlockSpec((R, 2 * E), lambda i: (i, 0)),
            full(1, B), full(1, B),
            pl.BlockSpec((R, 1), lambda i: (i, 0)),
            full(3 * E, H), full(1, H), full(1, H), full(1, H),
            full(H, H), full(1, H), full(1, H), full(1, H),
            full(H, E), full(1, E),
        ],
        out_specs=full(B, E),
        out_shape=jax.ShapeDtypeStruct((B, E), jnp.float32),
        scratch_shapes=[
            pltpu.VMEM((B, H), _bf),
            pltpu.VMEM((B, H), jnp.float32),
            pltpu.VMEM((B, 1), jnp.float32),
        ],
        compiler_params=pltpu.CompilerParams(
            dimension_semantics=("arbitrary",)),
    )(candidate_rep, graph_rep, starts, ends, put2d,
      W1, b1, g1, bb1, W2, b2, g2, bb2, W3, b3)


def kernel(candidate_rep, graph_rep, graph_sizes, put_indices,
           W1, b1, g1, bb1, W2, b2, g2, bb2, W3, b3):
    ends = jnp.cumsum(graph_sizes.astype(jnp.int32))
    starts = (ends - graph_sizes).reshape(1, B)
    ends = ends.reshape(1, B)
    put2d = put_indices.reshape(TOTAL, 1)

    def row(v):
        return v.reshape(1, -1)

    gout = _sc_copy(graph_rep)
    out = _run(candidate_rep, graph_rep.reshape(TOTAL, 2 * E),
               starts, ends, put2d,
               W1.astype(_bf), row(b1), row(g1), row(bb1),
               W2.astype(_bf), row(b2), row(g2), row(bb2),
               W3.astype(_bf), row(b3))
    return (out, gout)


# speedup vs baseline: 1.3389x; 1.3389x over previous
import functools

import jax
import jax.numpy as jnp
from jax.experimental import pallas as pl
from jax.experimental.pallas import tpu as pltpu
from jax.experimental.pallas import tpu_sc as plsc

B = 400
E = 256
TOTAL = B * (B - 1) // 2
H = 512
R = 2280
GRID = TOTAL // R

_bf = jnp.bfloat16


_SC_NW = 32
_SC_Q = 2496
_SC_C = 96


def _sc_copy_body(g_hbm, out_hbm, bufs, rsem, wsem):
    wid = jax.lax.axis_index("s") * 2 + jax.lax.axis_index("c")
    base = wid * _SC_Q
    nch = _SC_Q // _SC_C

    def off_of(k):
        return jnp.minimum(base + k * _SC_C, TOTAL - _SC_C)

    def rd(k, b):
        return pltpu.make_async_copy(
            g_hbm.at[pl.ds(off_of(k), _SC_C), :, :], bufs.at[b], rsem.at[b])

    def wr(k, b):
        return pltpu.make_async_copy(
            bufs.at[b], out_hbm.at[pl.ds(off_of(k), _SC_C), :, :], wsem.at[b])

    rd(0, 0).start()

    @pl.loop(0, nch)
    def _chunk(k):
        b = jax.lax.rem(k, 2)
        rd(k, b).wait()
        wr(k, b).start()

        @pl.when(k + 1 < nch)
        def _next():
            @pl.when(k >= 1)
            def _drain():
                wr(k - 1, 1 - b).wait()
            rd(k + 1, 1 - b).start()

    wr(nch - 2, jnp.int32(nch % 2)).wait()
    wr(nch - 1, jnp.int32((nch - 1) % 2)).wait()


def _sc_copy(graph_rep):
    return pl.kernel(
        _sc_copy_body,
        out_type=jax.ShapeDtypeStruct((TOTAL, 2, E), jnp.float32),
        mesh=plsc.VectorSubcoreMesh(core_axis_name="c", subcore_axis_name="s"),
        scratch_types=[
            pltpu.VMEM((2, _SC_C, 2, E), jnp.float32),
            pltpu.SemaphoreType.DMA((2,)),
            pltpu.SemaphoreType.DMA((2,)),
        ],
    )(graph_rep)


def _ln(x, g, b, eps=1e-5):
    mu = jnp.mean(x, axis=-1, keepdims=True)
    d = x - mu
    var = jnp.mean(d * d, axis=-1, keepdims=True)
    return d * jax.lax.rsqrt(var + eps) * g + b


def _mlp_kernel(cand_ref, g_ref, starts_ref, ends_ref, put_ref,
                w1_ref, b1_ref, g1_ref, bb1_ref,
                w2_ref, b2_ref, g2_ref, bb2_ref,
                w3_ref, b3_ref,
                out_ref,
                a_scr, acc_scr, cnt_scr):
    i = pl.program_id(0)

    @pl.when(i == 0)
    def _init():
        a_scr[...] = (jnp.dot(cand_ref[...].astype(_bf), w1_ref[0:E, :],
                              preferred_element_type=jnp.float32)
                      + b1_ref[...]).astype(_bf)
        acc_scr[...] = jnp.zeros_like(acc_scr)
        cnt_scr[...] = jnp.zeros_like(cnt_scr)

    iota_c = jax.lax.broadcasted_iota(jnp.int32, (B, 1), 0)
    t = i * R + jax.lax.broadcasted_iota(jnp.int32, (1, R), 1)
    ohT = jnp.logical_and(t >= starts_ref[...],
                          t < ends_ref[...]).astype(_bf)
    g2b = g_ref[...].astype(_bf).reshape(R, 2 * E)
    h = jax.lax.dot_general(ohT, a_scr[...], (((0,), (0,)), ((), ())),
                            preferred_element_type=jnp.float32)
    h += jnp.dot(g2b, w1_ref[E:, :], preferred_element_type=jnp.float32)
    h = jax.nn.relu(_ln(h, g1_ref[...], bb1_ref[...]))

    h = jnp.dot(h.astype(_bf), w2_ref[...],
                preferred_element_type=jnp.float32) + b2_ref[...]
    h = jax.nn.relu(_ln(h, g2_ref[...], bb2_ref[...]))

    ohT_put = (put_ref[0] == iota_c).astype(_bf)
    acc_scr[...] += jax.lax.dot_general(
        ohT_put, h.astype(_bf), (((1,), (0,)), ((), ())),
        preferred_element_type=jnp.float32)
    cnt_scr[...] += jax.lax.dot_general(
        ohT_put, jnp.ones((R, 1), _bf), (((1,), (0,)), ((), ())),
        preferred_element_type=jnp.float32)

    @pl.when(i == GRID - 1)
    def _fin():
        upd = jnp.dot(acc_scr[...].astype(_bf), w3_ref[...],
                      preferred_element_type=jnp.float32)
        out_ref[...] = cand_ref[...] + upd + cnt_scr[...] * b3_ref[...]


@jax.jit
def _run(candidate_rep, graph_rep, starts, ends, put2d,
         W1, b1, g1, bb1, W2, b2, g2, bb2, W3, b3):
    def full(*s):
        return pl.BlockSpec(s, lambda *_: tuple(0 for _ in s))
    return pl.pallas_call(
        _mlp_kernel,
        grid=(GRID,),
        in_specs=[
            full(B, E),
            pl.BlockSpec((R, 2, E), lambda i: (i, 0, 0)),
            full(B, 1), full(B, 1),
            pl.BlockSpec((1, 1, R), lambda i: (i, 0, 0)),
            full(3 * E, H), full(1, H), full(1, H), full(1, H),
            full(H, H), full(1, H), full(1, H), full(1, H),
            full(H, E), full(1, E),
        ],
        out_specs=full(B, E),
        out_shape=jax.ShapeDtypeStruct((B, E), jnp.float32),
        scratch_shapes=[
            pltpu.VMEM((B, H), _bf),
            pltpu.VMEM((B, H), jnp.float32),
            pltpu.VMEM((B, 1), jnp.float32),
        ],
        compiler_params=pltpu.CompilerParams(
            dimension_semantics=("arbitrary",)),
    )(candidate_rep, graph_rep, starts, ends, put2d,
      W1, b1, g1, bb1, W2, b2, g2, bb2, W3, b3)


def kernel(candidate_rep, graph_rep, graph_sizes, put_indices,
           W1, b1, g1, bb1, W2, b2, g2, bb2, W3, b3):
    ends = jnp.cumsum(graph_sizes.astype(jnp.int32))
    starts = (ends - graph_sizes).reshape(B, 1)
    ends = ends.reshape(B, 1)
    put2d = put_indices.reshape(GRID, 1, R)

    def row(v):
        return v.reshape(1, -1)

    gout = _sc_copy(graph_rep)
    out = _run(candidate_rep, graph_rep, starts, ends, put2d,
               W1.astype(_bf), row(b1), row(g1), row(bb1),
               W2.astype(_bf), row(b2), row(g2), row(bb2),
               W3.astype(_bf), row(b3))
    return (out, gout)
